# baseline (speedup 1.0000x reference)
import jax
import jax.numpy as jnp
from jax.experimental import pallas as pl
from jax.experimental.pallas import tpu as pltpu

T = 32768
D = 768
E = 64
TB = 1024


def _gate_body(x_ref, sim_ref, gates_ref, mask_ref, probs_ref, logits_ref):
    x = x_ref[...]
    sim = sim_ref[...]
    g = gates_ref[...]

    sn = sim / jnp.clip(
        jnp.sqrt(jnp.sum(sim * sim, axis=0, keepdims=True)), 1e-12
    )
    xn = x / jnp.clip(jnp.sqrt(jnp.sum(x * x, axis=1, keepdims=True)), 1e-12)

    logits = jnp.dot(xn, sn, preferred_element_type=jnp.float32)

    thr = 1.0 / (1.0 + jnp.exp(-g))
    gated = jnp.maximum(logits - thr, 0.0)
    mask = (gated > 0.0).astype(jnp.float32)

    inactive = jnp.sum(mask, axis=1, keepdims=True) == 0.0

    col = jax.lax.broadcasted_iota(jnp.int32, logits.shape, 1)
    rowmax = jnp.max(logits, axis=1, keepdims=True)
    idx = jnp.where(logits == rowmax, col, jnp.int32(E))
    top1 = jnp.min(idx, axis=1, keepdims=True)
    onehot = col == top1

    mask = jnp.where(inactive & onehot, 1.0, mask)

    gm = jnp.where(mask > 0.0, gated, jnp.float32(-1e9))
    m2 = jnp.max(gm, axis=1, keepdims=True)
    ex = jnp.exp(gm - m2)
    probs = ex / jnp.sum(ex, axis=1, keepdims=True)

    mask_ref[...] = mask
    probs_ref[...] = probs
    logits_ref[...] = logits


def kernel(x, sim_matrix, gates):
    gates2d = gates.reshape(1, E)
    out_shapes = (
        jax.ShapeDtypeStruct((T, E), jnp.float32),
        jax.ShapeDtypeStruct((T, E), jnp.float32),
        jax.ShapeDtypeStruct((T, E), jnp.float32),
    )
    grid = (T // TB,)
    mask, probs, logits = pl.pallas_call(
        _gate_body,
        grid=grid,
        in_specs=[
            pl.BlockSpec((TB, D), lambda i: (i, 0)),
            pl.BlockSpec((D, E), lambda i: (0, 0)),
            pl.BlockSpec((1, E), lambda i: (0, 0)),
        ],
        out_specs=(
            pl.BlockSpec((TB, E), lambda i: (i, 0)),
            pl.BlockSpec((TB, E), lambda i: (i, 0)),
            pl.BlockSpec((TB, E), lambda i: (i, 0)),
        ),
        out_shape=out_shapes,
        compiler_params=pltpu.CompilerParams(
            dimension_semantics=("arbitrary",),
        ),
    )(x, sim_matrix, gates2d)
    return (mask, probs, logits)


# speedup vs baseline: 1.0846x; 1.0846x over previous
import jax
import jax.numpy as jnp
from jax.experimental import pallas as pl
from jax.experimental.pallas import tpu as pltpu

T = 32768
D = 768
E = 64
TB = 2048


def _gate_body(x_ref, sim_ref, gates_ref, mask_ref, probs_ref, logits_ref):
    x = x_ref[...]
    sim = sim_ref[...]
    g = gates_ref[...]

    sn = sim / jnp.clip(
        jnp.sqrt(jnp.sum(sim * sim, axis=0, keepdims=True)), 1e-12
    )
    rnorm = jnp.maximum(jnp.sqrt(jnp.sum(x * x, axis=1, keepdims=True)), 1e-12)
    logits = jnp.dot(x, sn, preferred_element_type=jnp.float32) / rnorm

    thr = 1.0 / (1.0 + jnp.exp(-g))
    gated = jnp.maximum(logits - thr, 0.0)
    mask = (gated > 0.0).astype(jnp.float32)

    inactive = jnp.sum(mask, axis=1, keepdims=True) == 0.0

    col = jax.lax.broadcasted_iota(jnp.int32, logits.shape, 1)
    rowmax = jnp.max(logits, axis=1, keepdims=True)
    idx = jnp.where(logits == rowmax, col, jnp.int32(E))
    top1 = jnp.min(idx, axis=1, keepdims=True)
    onehot = col == top1

    mask = jnp.where(inactive & onehot, 1.0, mask)

    gm = jnp.where(mask > 0.0, gated, jnp.float32(-1e9))
    m2 = jnp.max(gm, axis=1, keepdims=True)
    ex = jnp.exp(gm - m2)
    probs = ex / jnp.sum(ex, axis=1, keepdims=True)

    mask_ref[...] = mask
    probs_ref[...] = probs
    logits_ref[...] = logits


def kernel(x, sim_matrix, gates):
    gates2d = gates.reshape(1, E)
    out_shapes = (
        jax.ShapeDtypeStruct((T, E), jnp.float32),
        jax.ShapeDtypeStruct((T, E), jnp.float32),
        jax.ShapeDtypeStruct((T, E), jnp.float32),
    )
    grid = (T // TB,)
    mask, probs, logits = pl.pallas_call(
        _gate_body,
        grid=grid,
        in_specs=[
            pl.BlockSpec((TB, D), lambda i: (i, 0)),
            pl.BlockSpec((D, E), lambda i: (0, 0)),
            pl.BlockSpec((1, E), lambda i: (0, 0)),
        ],
        out_specs=(
            pl.BlockSpec((TB, E), lambda i: (i, 0)),
            pl.BlockSpec((TB, E), lambda i: (i, 0)),
            pl.BlockSpec((TB, E), lambda i: (i, 0)),
        ),
        out_shape=out_shapes,
        compiler_params=pltpu.CompilerParams(
            dimension_semantics=("arbitrary",),
        ),
    )(x, sim_matrix, gates2d)
    return (mask, probs, logits)
